# Initial kernel scaffold; baseline (speedup 1.0000x reference)
import jax
import jax.numpy as jnp
from jax.experimental import pallas as pl


def kernel(user_item_id, item_feature, all_items, W, b, gamma, beta):
    raise NotImplementedError("write your pallas kernel here")


# speedup vs baseline: 2.0707x; 2.0707x over previous
import functools

import jax
import jax.numpy as jnp
from jax import lax
from jax.experimental import pallas as pl
from jax.experimental.pallas import tpu as pltpu

_KB = 1024


def _topk_body(feat_ref, w_ref, aux_ref, items_ref,
               idx_ref, val_ref,
               h_s, bval_s, bidx_s, *, n_b, n_k, nsteps, kb):
    k = pl.program_id(0)

    @pl.when(k == 0)
    def _init():
        b = aux_ref[0:1, 0:16]
        gamma = aux_ref[1:2, 0:16]
        beta = aux_ref[2:3, 0:16]
        h = lax.dot_general(feat_ref[...], w_ref[...],
                            (((1,), (1,)), ((), ())),
                            preferred_element_type=jnp.float32) + b
        mu = jnp.mean(h, axis=0, keepdims=True)
        var = jnp.mean((h - mu) ** 2, axis=0, keepdims=True)
        h = (h - mu) / jnp.sqrt(var + 1e-5)
        h = gamma * h + beta
        h = jnp.where(h >= 0, h, 0.01 * h)
        h_s[...] = h
        bval_s[...] = jnp.full((n_b, 1), -jnp.inf, jnp.float32)
        bidx_s[...] = jnp.zeros((n_b, 1), jnp.int32)

    score = lax.dot_general(h_s[...], items_ref[...],
                            (((1,), (1,)), ((), ())),
                            preferred_element_type=jnp.float32)

    def _update(s):
        m = jnp.max(s, axis=1, keepdims=True)
        a = jnp.argmax(s, axis=1).astype(jnp.int32).reshape(n_b, 1)
        better = m > bval_s[...]
        bval_s[...] = jnp.where(better, m, bval_s[...])
        bidx_s[...] = jnp.where(better, a + k * kb, bidx_s[...])

    @pl.when(k < nsteps - 1)
    def _main():
        _update(score)

    @pl.when(k == nsteps - 1)
    def _tail():
        colmask = lax.broadcasted_iota(jnp.int32, (1, kb), 1) < (n_k - k * kb)
        _update(jnp.where(colmask, score, -jnp.inf))
        idx_ref[...] = bidx_s[...]
        val_ref[...] = bval_s[...]


def _top1(item_feature, all_items, W, aux):
    n_b = item_feature.shape[0]
    n_k, d = all_items.shape
    nsteps = pl.cdiv(n_k, _KB)
    grid = (nsteps,)
    body = functools.partial(_topk_body, n_b=n_b, n_k=n_k,
                             nsteps=nsteps, kb=_KB)
    idx, val = pl.pallas_call(
        body,
        grid=grid,
        in_specs=[
            pl.BlockSpec((n_b, item_feature.shape[1]), lambda k: (0, 0)),
            pl.BlockSpec(W.shape, lambda k: (0, 0)),
            pl.BlockSpec(aux.shape, lambda k: (0, 0)),
            pl.BlockSpec((_KB, d), lambda k: (k, 0)),
        ],
        out_specs=[
            pl.BlockSpec((n_b, 1), lambda k: (0, 0)),
            pl.BlockSpec((n_b, 1), lambda k: (0, 0)),
        ],
        out_shape=[
            jax.ShapeDtypeStruct((n_b, 1), jnp.int32),
            jax.ShapeDtypeStruct((n_b, 1), jnp.float32),
        ],
        scratch_shapes=[
            pltpu.VMEM((n_b, d), jnp.float32),
            pltpu.VMEM((n_b, 1), jnp.float32),
            pltpu.VMEM((n_b, 1), jnp.int32),
        ],
    )(item_feature, W, aux, all_items)
    return idx, val


def kernel(user_item_id, item_feature, all_items, W, b, gamma, beta):
    n_b = item_feature.shape[0]
    aux = jnp.zeros((8, W.shape[1]), jnp.float32)
    aux = aux.at[0, :16].set(b).at[1, :16].set(gamma).at[2, :16].set(beta)

    idx, _ = _top1(item_feature, all_items, W, aux)
    sorted_items = idx.reshape(-1)

    original_items = user_item_id[:, 1]
    orig_feat = jnp.take(all_items, original_items, axis=0)
    sort_feat = jnp.take(all_items, sorted_items, axis=0)
    eps = 1e-6
    dot = jnp.sum(orig_feat * sort_feat, axis=1)
    na = jnp.sqrt(jnp.sum(orig_feat * orig_feat, axis=1))
    nc = jnp.sqrt(jnp.sum(sort_feat * sort_feat, axis=1))
    similarity = dot / (jnp.maximum(na, eps) * jnp.maximum(nc, eps))
    similarity = (similarity + 1.0) / 2.0
    similarity_loss = jnp.mean((similarity - 0.5) ** 2)
    return (sorted_items, similarity_loss, jnp.mean(similarity))
